# baseline (speedup 1.0000x reference)
import functools

import jax
import jax.numpy as jnp
from jax import lax
from jax.experimental import pallas as pl
from jax.experimental.pallas import tpu as pltpu
from jax.experimental.pallas import tpu_sc as plsc

_LANES = 16
_LN2 = 0.6931471805599453
_TINY = 1.1754943508222875e-38
_SCALE24 = 16777216.0


def _ln(x):
    tiny = x < _TINY
    xs = jnp.where(tiny, x * _SCALE24, x)
    bits = lax.bitcast_convert_type(xs, jnp.int32)
    e = lax.shift_right_logical(bits, 23) - 127 - jnp.where(tiny, 24, 0)
    m = lax.bitcast_convert_type(
        jnp.bitwise_or(jnp.bitwise_and(bits, 0x007FFFFF), 0x3F800000),
        jnp.float32)
    t = (m - 1.0) / (m + 1.0)
    t2 = t * t
    ln_m = t * (2.0 + t2 * (2.0 / 3.0 + t2 * (2.0 / 5.0 + t2 * (2.0 / 7.0))))
    ln_x = e.astype(jnp.float32) * _LN2 + ln_m
    return jnp.where(x == 0.0, -jnp.inf, ln_x)


def _sc_loss_partials(p1_flat, p2_flat, c_flat, t1_flat, t2_flat, v1, v2,
                      n_tok):
    info = plsc.get_sparse_core_info()
    nc, ns = info.num_cores, info.num_subcores
    nw = nc * ns
    chunk = n_tok // nw
    assert chunk % _LANES == 0

    mesh = plsc.VectorSubcoreMesh(core_axis_name="c", subcore_axis_name="s")

    @functools.partial(
        pl.kernel,
        out_type=jax.ShapeDtypeStruct((nw, 4, _LANES), jnp.float32),
        mesh=mesh,
        scratch_types=[
            pltpu.VMEM((chunk,), jnp.int32),
            pltpu.VMEM((chunk,), jnp.int32),
            pltpu.VMEM((chunk,), jnp.int32),
            pltpu.VMEM((chunk,), jnp.int32),
            pltpu.VMEM((chunk,), jnp.int32),
            pltpu.VMEM((chunk,), jnp.float32),
            pltpu.VMEM((chunk,), jnp.float32),
            pltpu.VMEM((chunk,), jnp.float32),
            pltpu.VMEM((4, _LANES), jnp.float32),
            pltpu.SemaphoreType.DMA,
            pltpu.SemaphoreType.DMA,
            pltpu.SemaphoreType.DMA,
            pltpu.SemaphoreType.DMA,
            pltpu.SemaphoreType.DMA,
        ],
    )
    def k(p1_hbm, p2_hbm, c_hbm, t1_hbm, t2_hbm, out_hbm,
          t1_v, t2_v, i1_v, i2_v, ic_v, r1_v, r2_v, rc_v,
          acc_v, sem1, sem2, sem3, sem4, sem5):
        cid = lax.axis_index("c")
        sid = lax.axis_index("s")
        wid = sid * nc + cid
        base = wid * chunk
        in_sl = pl.ds(base, chunk)
        ld1 = pltpu.async_copy(t1_hbm.at[in_sl], t1_v, sem4)
        ld2 = pltpu.async_copy(t2_hbm.at[in_sl], t2_v, sem5)
        iota = lax.broadcasted_iota(jnp.int32, (_LANES,), 0)
        ld1.wait()
        for i in range(chunk // _LANES):
            sl = pl.ds(i * _LANES, _LANES)
            t1 = t1_v[sl]
            s1 = jnp.where(t1 == -100, 0, t1)
            tok = base + i * _LANES + iota
            i1_v[sl] = (lax.shift_right_logical(tok, 3) * (v1 * 8)
                        + lax.shift_right_logical(s1, 7) * 1024
                        + jnp.bitwise_and(tok, 7) * 128
                        + jnp.bitwise_and(s1, 127))
        cp1 = pltpu.async_copy(p1_hbm.at[i1_v], r1_v, sem1)
        ld2.wait()
        for i in range(chunk // _LANES):
            sl = pl.ds(i * _LANES, _LANES)
            t2 = t2_v[sl]
            s2 = jnp.where(t2 == -100, 0, t2)
            tok = base + i * _LANES + iota
            i2_v[sl] = (lax.shift_right_logical(tok, 3) * (v2 * 8)
                        + lax.shift_right_logical(s2, 7) * 1024
                        + jnp.bitwise_and(tok, 7) * 128
                        + jnp.bitwise_and(s2, 127))
        cp2 = pltpu.async_copy(p2_hbm.at[i2_v], r2_v, sem2)
        for i in range(chunk // _LANES):
            sl = pl.ds(i * _LANES, _LANES)
            t1 = t1_v[sl]
            t2 = t2_v[sl]
            s1 = jnp.where(t1 == -100, 0, t1)
            s2 = jnp.where(t2 == -100, 0, t2)
            ic_v[sl] = (lax.shift_right_logical(s1, 3) * (v2 * 8)
                        + lax.shift_right_logical(s2, 7) * 1024
                        + jnp.bitwise_and(s1, 7) * 128
                        + jnp.bitwise_and(s2, 127))
        cp3 = pltpu.async_copy(c_hbm.at[ic_v], rc_v, sem3)
        cp1.wait()
        cp2.wait()
        cp3.wait()

        zero = jnp.zeros((_LANES,), jnp.float32)
        acc1 = zero
        acc2 = zero
        cnt1 = zero
        cnt2 = zero
        for i in range(chunk // _LANES):
            sl = pl.ds(i * _LANES, _LANES)
            c = jnp.clip(rc_v[sl], 0.0, 1.0)
            nl1 = -_ln(r1_v[sl] * c)
            nl2 = -_ln(r2_v[sl] * c)
            m1 = t1_v[sl] != -100
            m2 = t2_v[sl] != -100
            acc1 = acc1 + jnp.where(m1, nl1, 0.0)
            acc2 = acc2 + jnp.where(m2, nl2, 0.0)
            cnt1 = cnt1 + jnp.where(m1, 1.0, 0.0)
            cnt2 = cnt2 + jnp.where(m2, 1.0, 0.0)
        acc_v[0] = acc1
        acc_v[1] = cnt1
        acc_v[2] = acc2
        acc_v[3] = cnt2
        pltpu.sync_copy(acc_v, out_hbm.at[wid])

    return k(p1_flat, p2_flat, c_flat, t1_flat, t2_flat)


def _tile_order_view(x):
    r, c = x.shape
    return x.reshape(r // 8, 8, c // 128, 128).transpose(0, 2, 1, 3).reshape(-1)


def kernel(dec1_probs, dec2_probs, dec1_tgt, dec2_tgt, constrainer):
    b, l, v1 = dec1_probs.shape
    v2 = dec2_probs.shape[2]
    n_tok = b * l

    partials = _sc_loss_partials(
        _tile_order_view(dec1_probs.reshape(n_tok, v1)),
        _tile_order_view(dec2_probs.reshape(n_tok, v2)),
        _tile_order_view(constrainer),
        dec1_tgt.reshape(-1),
        dec2_tgt.reshape(-1),
        v1, v2, n_tok,
    )
    tot = partials.sum(axis=(0, 2))
    return tot[0] / jnp.maximum(tot[1], 1.0) + tot[2] / jnp.maximum(tot[3], 1.0)


# speedup vs baseline: 1.1519x; 1.1519x over previous
import functools

import jax
import jax.numpy as jnp
from jax import lax
from jax.experimental import pallas as pl
from jax.experimental.pallas import tpu as pltpu
from jax.experimental.pallas import tpu_sc as plsc

_LANES = 16


def _sc_gather(p1_flat, p2_flat, c_flat, t1_flat, t2_flat, v1, v2, n_tok):
    info = plsc.get_sparse_core_info()
    nc, ns = info.num_cores, info.num_subcores
    nw = nc * ns
    chunk = n_tok // nw
    assert chunk % _LANES == 0 and (chunk * 4) % 8 == 0

    mesh = plsc.VectorSubcoreMesh(core_axis_name="c", subcore_axis_name="s")

    @functools.partial(
        pl.kernel,
        out_type=[
            jax.ShapeDtypeStruct((n_tok,), jnp.float32),
            jax.ShapeDtypeStruct((n_tok,), jnp.float32),
        ],
        mesh=mesh,
        scratch_types=[
            pltpu.VMEM((chunk,), jnp.int32),
            pltpu.VMEM((chunk,), jnp.int32),
            pltpu.VMEM((chunk,), jnp.int32),
            pltpu.VMEM((chunk,), jnp.int32),
            pltpu.VMEM((chunk,), jnp.int32),
            pltpu.VMEM((chunk,), jnp.float32),
            pltpu.VMEM((chunk,), jnp.float32),
            pltpu.VMEM((chunk,), jnp.float32),
            pltpu.SemaphoreType.DMA,
            pltpu.SemaphoreType.DMA,
            pltpu.SemaphoreType.DMA,
            pltpu.SemaphoreType.DMA,
            pltpu.SemaphoreType.DMA,
        ],
    )
    def k(p1_hbm, p2_hbm, c_hbm, t1_hbm, t2_hbm,
          y1_hbm, y2_hbm,
          t1_v, t2_v, i1_v, i2_v, ic_v, r1_v, r2_v, rc_v,
          sem1, sem2, sem3, sem4, sem5):
        wid = lax.axis_index("s") * nc + lax.axis_index("c")
        base = wid * chunk
        in_sl = pl.ds(base, chunk)
        ld1 = pltpu.async_copy(t1_hbm.at[in_sl], t1_v, sem4)
        ld2 = pltpu.async_copy(t2_hbm.at[in_sl], t2_v, sem5)
        iota = lax.broadcasted_iota(jnp.int32, (_LANES,), 0)
        ld1.wait()
        for i in range(chunk // _LANES):
            sl = pl.ds(i * _LANES, _LANES)
            t1 = t1_v[sl]
            s1 = jnp.where(t1 == -100, 0, t1)
            tok = base + i * _LANES + iota
            i1_v[sl] = (lax.shift_right_logical(tok, 3) * (v1 * 8)
                        + lax.shift_right_logical(s1, 7) * 1024
                        + jnp.bitwise_and(tok, 7) * 128
                        + jnp.bitwise_and(s1, 127))
        cp1 = pltpu.async_copy(p1_hbm.at[i1_v], r1_v, sem1)
        ld2.wait()
        for i in range(chunk // _LANES):
            sl = pl.ds(i * _LANES, _LANES)
            t2 = t2_v[sl]
            s2 = jnp.where(t2 == -100, 0, t2)
            tok = base + i * _LANES + iota
            i2_v[sl] = (lax.shift_right_logical(tok, 3) * (v2 * 8)
                        + lax.shift_right_logical(s2, 7) * 1024
                        + jnp.bitwise_and(tok, 7) * 128
                        + jnp.bitwise_and(s2, 127))
        cp2 = pltpu.async_copy(p2_hbm.at[i2_v], r2_v, sem2)
        for i in range(chunk // _LANES):
            sl = pl.ds(i * _LANES, _LANES)
            t1 = t1_v[sl]
            t2 = t2_v[sl]
            s1 = jnp.where(t1 == -100, 0, t1)
            s2 = jnp.where(t2 == -100, 0, t2)
            ic_v[sl] = (lax.shift_right_logical(s1, 3) * (v2 * 8)
                        + lax.shift_right_logical(s2, 7) * 1024
                        + jnp.bitwise_and(s1, 7) * 128
                        + jnp.bitwise_and(s2, 127))
        cp3 = pltpu.async_copy(c_hbm.at[ic_v], rc_v, sem3)
        out_sl = pl.ds(base, chunk)
        cp1.wait()
        cp2.wait()
        cp3.wait()
        for i in range(chunk // _LANES):
            sl = pl.ds(i * _LANES, _LANES)
            c = jnp.clip(rc_v[sl], 0.0, 1.0)
            m1 = t1_v[sl] != -100
            m2 = t2_v[sl] != -100
            r1_v[sl] = jnp.where(m1, r1_v[sl] * c, 1.0)
            r2_v[sl] = jnp.where(m2, r2_v[sl] * c, 1.0)
        st1 = pltpu.async_copy(r1_v, y1_hbm.at[out_sl], sem4)
        st2 = pltpu.async_copy(r2_v, y2_hbm.at[out_sl], sem5)
        st1.wait()
        st2.wait()

    return k(p1_flat, p2_flat, c_flat, t1_flat, t2_flat)


def _tc_loss_body(y1_ref, y2_ref, t1_ref, t2_ref, out_ref):
    n1 = jnp.maximum(jnp.sum((t1_ref[...] != -100).astype(jnp.float32)), 1.0)
    n2 = jnp.maximum(jnp.sum((t2_ref[...] != -100).astype(jnp.float32)), 1.0)
    s1 = jnp.sum(-jnp.log(y1_ref[...]))
    s2 = jnp.sum(-jnp.log(y2_ref[...]))
    out_ref[0, 0] = s1 / n1 + s2 / n2


def _tile_order_view(x):
    r, c = x.shape
    return x.reshape(r // 8, 8, c // 128, 128).transpose(0, 2, 1, 3).reshape(-1)


def kernel(dec1_probs, dec2_probs, dec1_tgt, dec2_tgt, constrainer):
    b, l, v1 = dec1_probs.shape
    v2 = dec2_probs.shape[2]
    n_tok = b * l

    y1, y2 = _sc_gather(
        _tile_order_view(dec1_probs.reshape(n_tok, v1)),
        _tile_order_view(dec2_probs.reshape(n_tok, v2)),
        _tile_order_view(constrainer),
        dec1_tgt.reshape(-1),
        dec2_tgt.reshape(-1),
        v1, v2, n_tok,
    )

    out = pl.pallas_call(
        _tc_loss_body,
        out_shape=jax.ShapeDtypeStruct((1, 1), jnp.float32),
        out_specs=pl.BlockSpec(memory_space=pltpu.SMEM),
    )(y1, y2, dec1_tgt, dec2_tgt)
    return out[0, 0]
